# Initial kernel scaffold; baseline (speedup 1.0000x reference)
import jax
import jax.numpy as jnp
from jax.experimental import pallas as pl


def kernel(x, edge_index, Wl1, bl1, Wr1, br1, att1, b1, Wl2, bl2, Wr2, br2, att2, b2, Wl3, bl3, Wr3, br3, att3, b3):
    raise NotImplementedError("write your pallas kernel here")


# speedup vs baseline: 23.6632x; 23.6632x over previous
import functools

import jax
import jax.numpy as jnp
import numpy as np
from jax import lax
from jax.experimental import pallas as pl
from jax.experimental.pallas import tpu as pltpu
from jax.experimental.pallas import tpu_sc as plsc

N = 10000
D_IN = 128
HID = 64
ACC_W = 80

HEADS_L1 = ((0,), (16,), (32,), (48,))
HEADS_L23 = ((0, 16), (32, 48))


def _sel_const():
    g1 = np.zeros((64, 8), np.float32)
    for c in range(64):
        g1[c, c // 16] = 1.0
    g23 = np.zeros((64, 8), np.float32)
    for c in range(64):
        g23[c, c // 32] = 1.0
    s64 = np.zeros((ACC_W, 64), np.float32)
    s64[:64] = np.eye(64, dtype=np.float32)
    sden1 = np.zeros((ACC_W, 64), np.float32)
    for h in range(4):
        sden1[64 + h, 16 * h:16 * (h + 1)] = 1.0
    sden23 = np.zeros((ACC_W, 64), np.float32)
    sden23[64, 0:32] = 1.0
    sden23[65, 32:64] = 1.0
    return g1, g1.T.copy(), g23, g23.T.copy(), s64, sden1, sden23


_G1, _GT1, _G23, _GT23, _S64, _SDEN1, _SDEN23 = _sel_const()


def _proj_body(x_ref, wl_ref, bl_ref, wr_ref, br_ref, attf_ref, g_ref,
               xl_ref, xr_ref, t_ref):
    xb = x_ref[...]
    xl = lax.dot_general(xb, wl_ref[...], (((1,), (0,)), ((), ())),
                         preferred_element_type=jnp.float32) + bl_ref[...]
    xr = lax.dot_general(xb, wr_ref[...], (((1,), (0,)), ((), ())),
                         preferred_element_type=jnp.float32) + br_ref[...]
    s = xl + xr
    m = jnp.maximum(s, 0.2 * s) * attf_ref[...]
    t_ref[...] = lax.dot_general(m, g_ref[...], (((1,), (0,)), ((), ())),
                                 preferred_element_type=jnp.float32)
    xl_ref[...] = xl
    xr_ref[...] = xr


def _proj_call(x, wl, bl, wr, br, attf, g, blk=1000):
    n, d = x.shape
    fout = wl.shape[1]
    grid = (n // blk,)
    return pl.pallas_call(
        _proj_body,
        grid=grid,
        in_specs=[
            pl.BlockSpec((blk, d), lambda i: (i, 0)),
            pl.BlockSpec((d, fout), lambda i: (0, 0)),
            pl.BlockSpec((1, fout), lambda i: (0, 0)),
            pl.BlockSpec((d, fout), lambda i: (0, 0)),
            pl.BlockSpec((1, fout), lambda i: (0, 0)),
            pl.BlockSpec((1, fout), lambda i: (0, 0)),
            pl.BlockSpec((fout, 8), lambda i: (0, 0)),
        ],
        out_specs=[
            pl.BlockSpec((blk, fout), lambda i: (i, 0)),
            pl.BlockSpec((blk, fout), lambda i: (i, 0)),
            pl.BlockSpec((blk, 8), lambda i: (i, 0)),
        ],
        out_shape=[
            jax.ShapeDtypeStruct((n, fout), jnp.float32),
            jax.ShapeDtypeStruct((n, fout), jnp.float32),
            jax.ShapeDtypeStruct((n, 8), jnp.float32),
        ],
    )(x, wl, bl, wr, br, attf, g)


def _make_edge_kernel(n_acc, n_edges, heads, e_chunk, interpret=False):
    mesh = plsc.VectorSubcoreMesh(core_axis_name="c", subcore_axis_name="s",
                                  num_cores=2, num_subcores=16)
    nw = 32
    per_w = n_edges // nw
    n_chunks = per_w // e_chunk
    rows_per_tile = n_acc // 16
    seg_list = tuple(s for segs in heads for s in segs)

    @functools.partial(
        pl.kernel,
        out_type=jax.ShapeDtypeStruct((2, n_acc, ACC_W), jnp.float32),
        mesh=mesh,
        scratch_types=[
            pltpu.VMEM((e_chunk,), jnp.int32),
            pltpu.VMEM((e_chunk,), jnp.int32),
            pltpu.VMEM((e_chunk, 64), jnp.float32),
            pltpu.VMEM((e_chunk, 64), jnp.float32),
            pltpu.VMEM((e_chunk, ACC_W), jnp.float32),
            pltpu.VMEM((64,), jnp.float32),
            pltpu.VMEM_SHARED((n_acc, ACC_W), jnp.float32),
            pltpu.SemaphoreType.DMA,
        ],
        compiler_params=pltpu.CompilerParams(needs_layout_passes=False,
                                             use_tc_tiling_on_sc=False),
        interpret=interpret,
    )
    def edge_kernel(src_hbm, dst_hbm, xl_hbm, xr_hbm, attf_hbm, zeros_hbm,
                    out_hbm, srcv, dstv, xlv, xrv, valsv, attv, acc_sp, sem):
        cid = lax.axis_index("c")
        sid = lax.axis_index("s")
        wid = sid * 2 + cid
        pltpu.sync_copy(zeros_hbm,
                        acc_sp.at[pl.ds(sid * rows_per_tile, rows_per_tile)])
        pltpu.sync_copy(attf_hbm, attv)
        plsc.subcore_barrier()
        att_regs = {s: attv[pl.ds(s, 16)] for s in seg_list}
        lane = lax.iota(jnp.int32, 16)
        base_w = wid * per_w

        def edge_body(e, carry):
            pden = jnp.zeros((16,), jnp.float32)
            for h, segs in enumerate(heads):
                a = None
                xl_regs = []
                for s in segs:
                    xl_h = xlv[e, pl.ds(s, 16)]
                    xr_h = xrv[e, pl.ds(s, 16)]
                    sm = xl_h + xr_h
                    m = jnp.maximum(sm, 0.2 * sm) * att_regs[s]
                    part = jnp.sum(m)
                    a = part if a is None else a + part
                    xl_regs.append((s, xl_h))
                pvec = jnp.exp(jnp.full((16,), a, jnp.float32))
                for s, xl_h in xl_regs:
                    valsv[e, pl.ds(s, 16)] = pvec * xl_h
                pden = jnp.where(lane == h, pvec, pden)
            valsv[e, pl.ds(64, 16)] = pden
            return carry

        def chunk_body(ci, carry):
            base = base_w + ci * e_chunk
            pltpu.sync_copy(src_hbm.at[pl.ds(base, e_chunk)], srcv)
            pltpu.sync_copy(dst_hbm.at[pl.ds(base, e_chunk)], dstv)
            cp1 = pltpu.async_copy(xl_hbm.at[srcv], xlv, sem)
            cp2 = pltpu.async_copy(xr_hbm.at[dstv], xrv, sem)
            cp1.wait()
            cp2.wait()
            lax.fori_loop(0, e_chunk, edge_body, 0, unroll=2)
            pltpu.sync_copy(valsv, acc_sp.at[dstv], add=True)
            return carry

        lax.fori_loop(0, n_chunks, chunk_body, 0)
        plsc.subcore_barrier()
        pltpu.sync_copy(
            acc_sp.at[pl.ds(sid * rows_per_tile, rows_per_tile)],
            out_hbm.at[cid, pl.ds(sid * rows_per_tile, rows_per_tile)])

    return edge_kernel


def _epi1_body(acc_ref, xl1_ref, t1_ref, s64_ref, sden_ref, gt1_ref, b1_ref,
               wl23_ref, bl23_ref, wr23_ref, br23_ref, att23_ref, g23_ref,
               src23_ref, dst23_ref, t23_ref):
    acc = acc_ref[0] + acc_ref[1]
    pfull = lax.dot_general(jnp.exp(t1_ref[...]), gt1_ref[...],
                            (((1,), (0,)), ((), ())),
                            preferred_element_type=jnp.float32)
    num = lax.dot_general(acc, s64_ref[...], (((1,), (0,)), ((), ())),
                          preferred_element_type=jnp.float32) \
        + pfull * xl1_ref[...]
    den = lax.dot_general(acc, sden_ref[...], (((1,), (0,)), ((), ())),
                          preferred_element_type=jnp.float32) + pfull
    hb = jnp.maximum(num / den + b1_ref[...], 0.0)
    src23 = lax.dot_general(hb, wl23_ref[...], (((1,), (0,)), ((), ())),
                            preferred_element_type=jnp.float32) + bl23_ref[...]
    dst23 = lax.dot_general(hb, wr23_ref[...], (((1,), (0,)), ((), ())),
                            preferred_element_type=jnp.float32) + br23_ref[...]
    s = src23 + dst23
    m = jnp.maximum(s, 0.2 * s) * att23_ref[...]
    t23_ref[...] = lax.dot_general(m, g23_ref[...], (((1,), (0,)), ((), ())),
                                   preferred_element_type=jnp.float32)
    src23_ref[...] = src23
    dst23_ref[...] = dst23


def _epi1_call(acc1, xl1, t1, b1r, wl23, bl23, wr23, br23, att23, blk=1000):
    n = xl1.shape[0]
    grid = (n // blk,)
    c0 = lambda i: (0, 0)
    return pl.pallas_call(
        _epi1_body,
        grid=grid,
        in_specs=[
            pl.BlockSpec((2, blk, ACC_W), lambda i: (0, i, 0)),
            pl.BlockSpec((blk, 64), lambda i: (i, 0)),
            pl.BlockSpec((blk, 8), lambda i: (i, 0)),
            pl.BlockSpec((ACC_W, 64), c0),
            pl.BlockSpec((ACC_W, 64), c0),
            pl.BlockSpec((8, 64), c0),
            pl.BlockSpec((1, 64), c0),
            pl.BlockSpec((64, 64), c0),
            pl.BlockSpec((1, 64), c0),
            pl.BlockSpec((64, 64), c0),
            pl.BlockSpec((1, 64), c0),
            pl.BlockSpec((1, 64), c0),
            pl.BlockSpec((64, 8), c0),
        ],
        out_specs=[
            pl.BlockSpec((blk, 64), lambda i: (i, 0)),
            pl.BlockSpec((blk, 64), lambda i: (i, 0)),
            pl.BlockSpec((blk, 8), lambda i: (i, 0)),
        ],
        out_shape=[
            jax.ShapeDtypeStruct((n, 64), jnp.float32),
            jax.ShapeDtypeStruct((n, 64), jnp.float32),
            jax.ShapeDtypeStruct((n, 8), jnp.float32),
        ],
    )(acc1, xl1, t1, jnp.asarray(_S64), jnp.asarray(_SDEN1),
      jnp.asarray(_GT1), b1r, wl23, bl23, wr23, br23, att23,
      jnp.asarray(_G23))


def _epi2_body(acc_ref, src23_ref, t23_ref, s64_ref, sden_ref, gt23_ref,
               b23_ref, out_ref):
    acc = acc_ref[0] + acc_ref[1]
    pfull = lax.dot_general(jnp.exp(t23_ref[...]), gt23_ref[...],
                            (((1,), (0,)), ((), ())),
                            preferred_element_type=jnp.float32)
    num = lax.dot_general(acc, s64_ref[...], (((1,), (0,)), ((), ())),
                          preferred_element_type=jnp.float32) \
        + pfull * src23_ref[...]
    den = lax.dot_general(acc, sden_ref[...], (((1,), (0,)), ((), ())),
                          preferred_element_type=jnp.float32) + pfull
    out_ref[...] = num / den + b23_ref[...]


def _epi2_call(acc23, src23, t23, b23, blk=1000):
    n = src23.shape[0]
    grid = (n // blk,)
    c0 = lambda i: (0, 0)
    return pl.pallas_call(
        _epi2_body,
        grid=grid,
        in_specs=[
            pl.BlockSpec((2, blk, ACC_W), lambda i: (0, i, 0)),
            pl.BlockSpec((blk, 64), lambda i: (i, 0)),
            pl.BlockSpec((blk, 8), lambda i: (i, 0)),
            pl.BlockSpec((ACC_W, 64), c0),
            pl.BlockSpec((ACC_W, 64), c0),
            pl.BlockSpec((8, 64), c0),
            pl.BlockSpec((1, 64), c0),
        ],
        out_specs=pl.BlockSpec((blk, 64), lambda i: (i, 0)),
        out_shape=jax.ShapeDtypeStruct((n, 64), jnp.float32),
    )(acc23, src23, t23, jnp.asarray(_S64), jnp.asarray(_SDEN23),
      jnp.asarray(_GT23), b23)


def kernel(x, edge_index, Wl1, bl1, Wr1, br1, att1, b1,
           Wl2, bl2, Wr2, br2, att2, b2,
           Wl3, bl3, Wr3, br3, att3, b3):
    n = x.shape[0]
    n_acc = ((n + 127) // 128) * 128
    e = edge_index.shape[1]
    src = edge_index[0]
    dst = edge_index[1]
    att1f = att1.reshape(1, -1).astype(jnp.float32)
    att23 = jnp.concatenate([att2.reshape(-1), att3.reshape(-1)]
                            ).reshape(1, -1).astype(jnp.float32)
    zeros = jnp.zeros((n_acc // 16, ACC_W), jnp.float32)

    xl1, xr1, t1 = _proj_call(x, Wl1, bl1.reshape(1, -1), Wr1,
                              br1.reshape(1, -1), att1f, jnp.asarray(_G1))

    ek1 = _make_edge_kernel(n_acc, e, HEADS_L1, e_chunk=80)
    acc1 = ek1(src, dst, xl1, xr1, att1f.reshape(-1), zeros)

    wl23 = jnp.concatenate([Wl2, Wl3], axis=1)
    wr23 = jnp.concatenate([Wr2, Wr3], axis=1)
    bl23 = jnp.concatenate([bl2, bl3]).reshape(1, -1)
    br23 = jnp.concatenate([br2, br3]).reshape(1, -1)
    src23, dst23, t23 = _epi1_call(acc1, xl1, t1, b1.reshape(1, -1),
                                   wl23, bl23, wr23, br23, att23)

    ek23 = _make_edge_kernel(n_acc, e, HEADS_L23, e_chunk=80)
    acc23 = ek23(src, dst, src23, dst23, att23.reshape(-1), zeros)

    b23 = jnp.concatenate([b2, b3]).reshape(1, -1)
    out64 = _epi2_call(acc23, src23, t23, b23)
    return out64[:, :32], out64[:, 32:]
